# Initial kernel scaffold; baseline (speedup 1.0000x reference)
import jax
import jax.numpy as jnp
from jax import lax
from jax.experimental import pallas as pl
from jax.experimental.pallas import tpu as pltpu
from jax.experimental.pallas import tpu_sc as plsc

NUM_CORES = 2
NUM_SUBCORES = 16
NW = NUM_CORES * NUM_SUBCORES

BATCH = 16384
LEVELS = 7
DIM = 64
ROWS = BATCH * LEVELS
RPW = ROWS // NW
CHUNK = 128
NCHUNK = RPW // CHUNK


def _body(paths_ref, table_ref, out_ref, pbuf, ibuf, gbuf, gsem):
    wid = lax.axis_index("s") * NUM_CORES + lax.axis_index("c")
    base = wid * NCHUNK

    pltpu.sync_copy(paths_ref.at[pl.ds(base, NCHUNK)], pbuf)

    def compute_idx(i, carry):
        c = i // 8
        j = (i % 8) * 16
        p = pbuf[c, pl.ds(j, 16)]
        k = i * 16 + lax.iota(jnp.int32, 16)
        r = lax.rem(k, jnp.full((16,), LEVELS, jnp.int32))
        off = lax.shift_left(jnp.full((16,), 4, jnp.int32), r) - 4
        ibuf[c, pl.ds(j, 16)] = p + off
        return carry

    lax.fori_loop(0, NCHUNK * 8, compute_idx, 0)

    def gather_chunk(c, carry):
        pltpu.async_copy(table_ref.at[ibuf.at[c]], gbuf, gsem).wait()
        pltpu.sync_copy(gbuf, out_ref.at[pl.ds(wid * RPW + c * CHUNK, CHUNK)])
        return carry

    lax.fori_loop(0, NCHUNK, gather_chunk, 0)


@jax.jit
def kernel(paths, W0, W1, W2, W3, W4, W5, W6):
    table = jnp.concatenate([W0, W1, W2, W3, W4, W5, W6], axis=0)
    paths2 = paths.reshape(ROWS // CHUNK, CHUNK)

    mesh = plsc.VectorSubcoreMesh(core_axis_name="c", subcore_axis_name="s")
    out = pl.kernel(
        _body,
        out_type=jax.ShapeDtypeStruct((ROWS, DIM), jnp.float32),
        mesh=mesh,
        scratch_types=[
            pltpu.VMEM((NCHUNK, CHUNK), jnp.int32),
            pltpu.VMEM((NCHUNK, CHUNK), jnp.int32),
            pltpu.VMEM((CHUNK, DIM), jnp.float32),
            pltpu.SemaphoreType.DMA,
        ],
    )(paths2, table)
    return out.reshape(BATCH, LEVELS * DIM)


# speedup vs baseline: 1.7155x; 1.7155x over previous
import jax
import jax.numpy as jnp
from jax import lax
from jax.experimental import pallas as pl
from jax.experimental.pallas import tpu as pltpu
from jax.experimental.pallas import tpu_sc as plsc

NUM_CORES = 2
NUM_SUBCORES = 16
NW = NUM_CORES * NUM_SUBCORES

BATCH = 16384
LEVELS = 7
DIM = 64
ROWS = BATCH * LEVELS
RPW = ROWS // NW
CHUNK = 128
NCHUNK = RPW // CHUNK


def _body(paths_ref, table_ref, out_ref, pbuf, ibuf, gbuf, gsem):
    wid = lax.axis_index("s") * NUM_CORES + lax.axis_index("c")

    pltpu.sync_copy(paths_ref.at[pl.ds(wid * RPW, RPW)], pbuf)

    def compute_idx(i, carry):
        c = i // 8
        j = (i % 8) * 16
        p = pbuf[pl.ds(i * 16, 16)]
        k = i * 16 + lax.iota(jnp.int32, 16)
        r = lax.rem(k, jnp.full((16,), LEVELS, jnp.int32))
        off = lax.shift_left(jnp.full((16,), 4, jnp.int32), r) - 4
        ibuf[c, pl.ds(j, 16)] = p + off
        return carry

    lax.fori_loop(0, NCHUNK * 8, compute_idx, 0)

    def gather_chunk(c, carry):
        pltpu.async_copy(table_ref.at[ibuf.at[c]], gbuf, gsem).wait()
        pltpu.sync_copy(gbuf, out_ref.at[pl.ds(wid * RPW + c * CHUNK, CHUNK)])
        return carry

    lax.fori_loop(0, NCHUNK, gather_chunk, 0)


@jax.jit
def kernel(paths, W0, W1, W2, W3, W4, W5, W6):
    table = jnp.concatenate([W0, W1, W2, W3, W4, W5, W6], axis=0)
    paths_flat = paths.reshape(ROWS)

    mesh = plsc.VectorSubcoreMesh(core_axis_name="c", subcore_axis_name="s")
    out = pl.kernel(
        _body,
        out_type=jax.ShapeDtypeStruct((ROWS, DIM), jnp.float32),
        mesh=mesh,
        compiler_params=pltpu.CompilerParams(use_tc_tiling_on_sc=False),
        scratch_types=[
            pltpu.VMEM((RPW,), jnp.int32),
            pltpu.VMEM((NCHUNK, CHUNK), jnp.int32),
            pltpu.VMEM((CHUNK, DIM), jnp.float32),
            pltpu.SemaphoreType.DMA,
        ],
    )(paths_flat, table)
    return out.reshape(BATCH, LEVELS * DIM)


# speedup vs baseline: 1.7237x; 1.0048x over previous
import jax
import jax.numpy as jnp
from jax import lax
from jax.experimental import pallas as pl
from jax.experimental.pallas import tpu as pltpu
from jax.experimental.pallas import tpu_sc as plsc

NUM_CORES = 2
NUM_SUBCORES = 16
NW = NUM_CORES * NUM_SUBCORES

BATCH = 16384
LEVELS = 7
DIM = 64
ROWS = BATCH * LEVELS
RPW = ROWS // NW
CHUNK = 128
NCHUNK = RPW // CHUNK


NBUF = 8
DEPTH = 6


def _body(paths_ref, table_ref, out_ref, pbuf, ibuf, bufs, gsem, ssem):
    wid = lax.axis_index("s") * NUM_CORES + lax.axis_index("c")

    pltpu.sync_copy(paths_ref.at[pl.ds(wid * RPW, RPW)], pbuf)

    def compute_idx(i, carry):
        c = i // 8
        j = (i % 8) * 16
        p = pbuf[pl.ds(i * 16, 16)]
        k = i * 16 + lax.iota(jnp.int32, 16)
        r = lax.rem(k, jnp.full((16,), LEVELS, jnp.int32))
        off = lax.shift_left(jnp.full((16,), 4, jnp.int32), r) - 4
        ibuf[c, pl.ds(j, 16)] = p + off
        return carry

    lax.fori_loop(0, NCHUNK * 8, compute_idx, 0)

    g_copy = {}
    s_copy = {}

    def start_gather(c):
        b = c % NBUF
        if c >= NBUF:
            s_copy[c - NBUF].wait()
        g_copy[c] = pltpu.async_copy(
            table_ref.at[ibuf.at[c]], bufs.at[b], gsem.at[b])

    def start_scatter(c):
        b = c % NBUF
        g_copy[c].wait()
        s_copy[c] = pltpu.async_copy(
            bufs.at[b],
            out_ref.at[pl.ds(wid * RPW + c * CHUNK, CHUNK)],
            ssem.at[b])

    for c in range(NCHUNK):
        start_gather(c)
        if c >= DEPTH - 1:
            start_scatter(c - (DEPTH - 1))
    for c in range(NCHUNK - DEPTH + 1, NCHUNK):
        start_scatter(c)
    for c in range(NCHUNK - NBUF, NCHUNK):
        s_copy[c].wait()


@jax.jit
def kernel(paths, W0, W1, W2, W3, W4, W5, W6):
    table = jnp.concatenate([W0, W1, W2, W3, W4, W5, W6], axis=0)
    paths_flat = paths.reshape(ROWS)

    mesh = plsc.VectorSubcoreMesh(core_axis_name="c", subcore_axis_name="s")
    out = pl.kernel(
        _body,
        out_type=jax.ShapeDtypeStruct((ROWS, DIM), jnp.float32),
        mesh=mesh,
        compiler_params=pltpu.CompilerParams(use_tc_tiling_on_sc=False),
        scratch_types=[
            pltpu.VMEM((RPW,), jnp.int32),
            pltpu.VMEM((NCHUNK, CHUNK), jnp.int32),
            pltpu.VMEM((NBUF, CHUNK, DIM), jnp.float32),
            pltpu.SemaphoreType.DMA((NBUF,)),
            pltpu.SemaphoreType.DMA((NBUF,)),
        ],
    )(paths_flat, table)
    return out.reshape(BATCH, LEVELS * DIM)
